# baseline (speedup 1.0000x reference)
import functools

import jax
import jax.numpy as jnp
from jax import lax
from jax.experimental import pallas as pl
from jax.experimental.pallas import tpu as pltpu
from jax.experimental.pallas import tpu_sc as plsc

_LANES = 16


def _gather_sums(ids, cids, W_dae, Wc):
    B, L = ids.shape
    _, Lc = cids.shape
    N, E = W_dae.shape
    info = plsc.get_sparse_core_info()
    NC, NS = info.num_cores, info.num_subcores
    NW = NC * NS
    RB = B // NW

    mesh = plsc.VectorSubcoreMesh(core_axis_name="c", subcore_axis_name="s")

    @functools.partial(
        pl.kernel,
        out_type=[
            jax.ShapeDtypeStruct((B, E), jnp.float32),
            jax.ShapeDtypeStruct((B, E), jnp.float32),
        ],
        mesh=mesh,
        compiler_params=pltpu.CompilerParams(use_tc_tiling_on_sc=False),
        scratch_types=[
            pltpu.VMEM((RB, L), jnp.int32),
            pltpu.VMEM((RB, Lc), jnp.int32),
            pltpu.VMEM((RB, L, E), jnp.float32),
            pltpu.VMEM((RB, Lc, E), jnp.float32),
            pltpu.VMEM((RB, E), jnp.float32),
            pltpu.VMEM((RB, E), jnp.float32),
            pltpu.SemaphoreType.DMA,
            pltpu.SemaphoreType.DMA,
        ],
    )
    def k(ids_hbm, cids_hbm, wdae_hbm, wc_hbm, out_i, out_c,
          idx_i, idx_c, rows_i, rows_c, acc_i, acc_c, sem_i, sem_c):
        wid = lax.axis_index("s") * NC + lax.axis_index("c")
        base = wid * RB
        pltpu.sync_copy(ids_hbm.at[pl.ds(base, RB)], idx_i)
        pltpu.sync_copy(cids_hbm.at[pl.ds(base, RB)], idx_c)
        cps = []
        for b in range(RB):
            cps.append(pltpu.async_copy(wdae_hbm.at[idx_i.at[b]], rows_i.at[b], sem_i))
            cps.append(pltpu.async_copy(wc_hbm.at[idx_c.at[b]], rows_c.at[b], sem_c))
        for cp in cps:
            cp.wait()

        nh = E // _LANES

        def body(b, _):
            for h in range(nh):
                sl = pl.ds(h * _LANES, _LANES)
                a = jnp.zeros((_LANES,), jnp.float32)
                for j in range(L):
                    a = a + rows_i[b, j, sl]
                acc_i[b, sl] = a
                a = jnp.zeros((_LANES,), jnp.float32)
                for j in range(Lc):
                    a = a + rows_c[b, j, sl]
                acc_c[b, sl] = a
            return _

        lax.fori_loop(0, RB, body, None)
        pltpu.sync_copy(acc_i, out_i.at[pl.ds(base, RB)])
        pltpu.sync_copy(acc_c, out_c.at[pl.ds(base, RB)])

    return k(ids, cids, W_dae, Wc)


def _dae_proj(W_dae, Wd1):
    N, E = W_dae.shape
    D = Wd1.shape[1]
    RT = 4
    R = N // RT

    def body(w_ref, wd_ref, out_ref):
        i = pl.program_id(0)

        @pl.when(i == 0)
        def _():
            out_ref[...] = jnp.zeros_like(out_ref)

        out_ref[...] += lax.dot_general(
            w_ref[...], wd_ref[...], (((0,), (0,)), ((), ())),
            preferred_element_type=jnp.float32)

    return pl.pallas_call(
        body,
        grid=(RT,),
        in_specs=[
            pl.BlockSpec((R, E), lambda i: (i, 0)),
            pl.BlockSpec((R, D), lambda i: (i, 0)),
        ],
        out_specs=pl.BlockSpec((E, D), lambda i: (0, 0)),
        out_shape=jax.ShapeDtypeStruct((E, D), jnp.float32),
    )(W_dae, Wd1)


_COLS = 2048


def _stats(s_dae, s_cnn, M, Wc1, bd1, bc1, Wf, bf2):
    B, E = s_dae.shape
    N = Wf.shape[1]
    H = Wf.shape[0]
    NT = pl.cdiv(N, _COLS)

    def body(sd_ref, sc_ref, m_ref, wc1_ref, bd1_ref, bc1_ref, wf_ref, bf_ref,
             h_ref, mx_ref, sm_ref):
        j = pl.program_id(0)

        @pl.when(j == 0)
        def _():
            x = jnp.maximum(sd_ref[...], 0.0)
            y_dae = jnp.maximum(
                jnp.dot(x, m_ref[...], preferred_element_type=jnp.float32)
                + bd1_ref[...], 0.0)
            t = jnp.maximum(
                jnp.dot(sc_ref[...], wc1_ref[...],
                        preferred_element_type=jnp.float32) + bc1_ref[...], 0.0)
            t = t - jnp.max(t, axis=1, keepdims=True)
            e = jnp.exp(t)
            y_cnn = e / jnp.sum(e, axis=1, keepdims=True)
            h_ref[...] = jnp.concatenate([y_dae, y_cnn], axis=1)
            mx_ref[...] = jnp.zeros_like(mx_ref)
            sm_ref[...] = jnp.zeros_like(sm_ref)

        logits = jnp.maximum(
            jnp.dot(h_ref[...], wf_ref[...], preferred_element_type=jnp.float32)
            + bf_ref[...], 0.0)
        col = j * _COLS + lax.broadcasted_iota(jnp.int32, (B, _COLS), 1)
        y = jnp.where(col < N, logits, -jnp.inf)
        tmax = jnp.max(y, axis=1, keepdims=True)
        m_old = mx_ref[...]
        m_new = jnp.maximum(m_old, tmax)
        p = jnp.exp(y - m_new)
        sm_ref[...] = sm_ref[...] * jnp.exp(m_old - m_new) + jnp.sum(
            p, axis=1, keepdims=True)
        mx_ref[...] = m_new

    return pl.pallas_call(
        body,
        grid=(NT,),
        in_specs=[
            pl.BlockSpec((B, E), lambda j: (0, 0)),
            pl.BlockSpec((B, E), lambda j: (0, 0)),
            pl.BlockSpec(M.shape, lambda j: (0, 0)),
            pl.BlockSpec(Wc1.shape, lambda j: (0, 0)),
            pl.BlockSpec((1, E), lambda j: (0, 0)),
            pl.BlockSpec((1, E), lambda j: (0, 0)),
            pl.BlockSpec((H, _COLS), lambda j: (0, j)),
            pl.BlockSpec((1, _COLS), lambda j: (0, j)),
        ],
        out_specs=[
            pl.BlockSpec((B, 2 * E), lambda j: (0, 0)),
            pl.BlockSpec((B, 1), lambda j: (0, 0)),
            pl.BlockSpec((B, 1), lambda j: (0, 0)),
        ],
        out_shape=[
            jax.ShapeDtypeStruct((B, 2 * E), jnp.float32),
            jax.ShapeDtypeStruct((B, 1), jnp.float32),
            jax.ShapeDtypeStruct((B, 1), jnp.float32),
        ],
    )(s_dae, s_cnn, M, Wc1, bd1, bc1, Wf, bf2)


def _head_out(h, mx, sm, Wf, bf2):
    B = h.shape[0]
    H, N = Wf.shape
    NT = pl.cdiv(N, _COLS)

    def body(h_ref, mx_ref, sm_ref, wf_ref, bf_ref, o_ref):
        y = jnp.maximum(
            jnp.dot(h_ref[...], wf_ref[...], preferred_element_type=jnp.float32)
            + bf_ref[...], 0.0)
        o_ref[...] = jnp.exp(y - mx_ref[...]) / sm_ref[...]

    return pl.pallas_call(
        body,
        grid=(NT,),
        in_specs=[
            pl.BlockSpec((B, H), lambda j: (0, 0)),
            pl.BlockSpec((B, 1), lambda j: (0, 0)),
            pl.BlockSpec((B, 1), lambda j: (0, 0)),
            pl.BlockSpec((H, _COLS), lambda j: (0, j)),
            pl.BlockSpec((1, _COLS), lambda j: (0, j)),
        ],
        out_specs=pl.BlockSpec((B, _COLS), lambda j: (0, j)),
        out_shape=jax.ShapeDtypeStruct((B, N), jnp.float32),
    )(h, mx, sm, Wf, bf2)


def kernel(ids, cids, W_dae, Wd1, bd1, Wc, Wc1, bc1, Wf, bf):
    ids = ids.astype(jnp.int32)
    cids = cids.astype(jnp.int32)
    s_dae, s_cnn = _gather_sums(ids, cids, W_dae, Wc)
    M = _dae_proj(W_dae, Wd1)
    h, mx, sm = _stats(s_dae, s_cnn, M, Wc1,
                       bd1.reshape(1, -1), bc1.reshape(1, -1),
                       Wf, bf.reshape(1, -1))
    return _head_out(h, mx, sm, Wf, bf.reshape(1, -1))
